# baseline (speedup 1.0000x reference)
import functools

import jax
import jax.numpy as jnp
from jax import lax
from jax.experimental import pallas as pl
from jax.experimental.pallas import tpu as pltpu
from jax.experimental.pallas import tpu_sc as plsc

B = 4
S = 2048
D = 768
L = 16
NV = D // L

_info = plsc.get_sparse_core_info()
NC = _info.num_cores
NS = _info.num_subcores
NW = NC * NS
PPW = S // NW

RPC = 16
CH = (B * PPW) // RPC
NBUF = 6
LOOK = 3
NPS = PPW // RPC
UNJ = 8
JB = 8
DB = D // (JB * L)

_GATHER_DNUMS = lax.GatherDimensionNumbers(
    offset_dims=(), collapsed_slice_dims=(0,), start_index_map=(0,))


def _xlane(x, pm):
    return lax.gather(x, pm[:, None], _GATHER_DNUMS, slice_sizes=(1,),
                      mode=lax.GatherScatterMode.PROMISE_IN_BOUNDS)


def _rsqrt(x):
    iv = lax.bitcast_convert_type(x, jnp.int32)
    iv = jnp.int32(0x5F3759DF) - lax.shift_right_logical(iv, 1)
    y = lax.bitcast_convert_type(iv, jnp.float32)
    for _ in range(3):
        y = y * (1.5 - 0.5 * x * y * y)
    return y


def _tree16(vs, lane):
    level = list(vs)
    for k in range(4):
        bit = 1 << k
        pm = lane ^ bit
        mk = (lane & bit) != 0
        nxt = []
        for j in range(len(level) // 2):
            a, b = level[2 * j], level[2 * j + 1]
            c = (jnp.where(mk, _xlane(b, pm), a)
                 + jnp.where(mk, b, _xlane(a, pm)))
            nxt.append(c)
        level = nxt
    return level[0]


def _make_kernel():
    mesh = plsc.VectorSubcoreMesh(core_axis_name="c", subcore_axis_name="s")

    @functools.partial(
        pl.kernel,
        mesh=mesh,
        out_type=jax.ShapeDtypeStruct((B, S, D), jnp.float32),
        scratch_types=[
            pltpu.VMEM((B, PPW), jnp.int32),
            [pltpu.VMEM((RPC, D), jnp.float32) for _ in range(2)],
            [pltpu.VMEM((RPC, D), jnp.float32) for _ in range(NBUF)],
            pltpu.VMEM((RPC, D), jnp.float32),
            pltpu.VMEM((D,), jnp.float32),
            pltpu.VMEM((D,), jnp.float32),
            pltpu.VMEM((L * L,), jnp.float32),
            pltpu.VMEM((L * L,), jnp.float32),
            pltpu.VMEM((L * L,), jnp.float32),
            pltpu.VMEM((L * L,), jnp.float32),
            [pltpu.SemaphoreType.DMA for _ in range(NBUF)],
            [pltpu.SemaphoreType.DMA for _ in range(NBUF)],
            [pltpu.SemaphoreType.DMA for _ in range(2)],
            pltpu.SemaphoreType.DMA,
            pltpu.SemaphoreType.DMA,
            pltpu.SemaphoreType.DMA,
        ],
    )
    def emb_ln(x_hbm, w_hbm, p_hbm, g_hbm, be_hbm, out_hbm,
               idx_v, pbufs, rows, hbuf, g_v, be_v, accb, acc2b, mv2, yv2,
               gsem, osem, psem, isem, sgsem, sbsem):
        wid = lax.axis_index("s") * NC + lax.axis_index("c")
        pos0 = wid * PPW

        cp_i = [pltpu.async_copy(x_hbm.at[b, pl.ds(pos0, PPW)],
                                 idx_v.at[b], isem) for b in range(B)]
        cp_p = [None, None]
        cp_p[0] = pltpu.async_copy(p_hbm.at[pl.ds(pos0, RPC), :],
                                   pbufs[0], psem[0])
        cp_g = pltpu.async_copy(g_hbm, g_v, sgsem)
        cp_b = pltpu.async_copy(be_hbm, be_v, sbsem)

        lane = lax.iota(jnp.int32, L)
        zero = jnp.zeros((L,), jnp.float32)

        def pass1(buf, pbuf):
            def pair_body(i, carry):
                rb = 2 * i

                def jblk_body(jc, accs):
                    (a00, a01, a10, a11, b00, b01, b10, b11) = accs
                    acc = [[a00, a01], [a10, a11]]
                    acc2 = [[b00, b01], [b10, b11]]
                    for jj in range(UNJ):
                        sl = pl.ds(jc * (UNJ * L) + jj * L, L)
                        for rr in range(2):
                            v = buf[rb + rr, sl] + pbuf[rb + rr, sl]
                            hbuf[rb + rr, sl] = v
                            a = jj % 2
                            acc[rr][a] = acc[rr][a] + v
                            acc2[rr][a] = acc2[rr][a] + v * v
                    return (acc[0][0], acc[0][1], acc[1][0], acc[1][1],
                            acc2[0][0], acc2[0][1], acc2[1][0], acc2[1][1])

                accs = lax.fori_loop(0, NV // UNJ, jblk_body, (zero,) * 8)
                accb[pl.ds(2 * i * L, L)] = accs[0] + accs[1]
                accb[pl.ds((2 * i + 1) * L, L)] = accs[2] + accs[3]
                acc2b[pl.ds(2 * i * L, L)] = accs[4] + accs[5]
                acc2b[pl.ds((2 * i + 1) * L, L)] = accs[6] + accs[7]
                return carry

            lax.fori_loop(0, L // 2, pair_body, 0)
            w = _tree16([accb[pl.ds(t * L, L)] for t in range(L)], lane)
            w2 = _tree16([acc2b[pl.ds(t * L, L)] for t in range(L)], lane)
            mean = w * (1.0 / D)
            var = w2 * (1.0 / D) - mean * mean
            y = _rsqrt(var + 1e-5)
            for t in range(L):
                pm = jnp.full((L,), t, jnp.int32)
                mv2[pl.ds(t * L, L)] = _xlane(mean, pm)
                yv2[pl.ds(t * L, L)] = _xlane(y, pm)

        def pass2(buf):
            def dblk_body(dblk, carry):
                d0 = dblk * (JB * L)
                gs = [g_v[pl.ds(d0 + j * L, L)] for j in range(JB)]
                bs = [be_v[pl.ds(d0 + j * L, L)] for j in range(JB)]

                def row_body(r, c2):
                    m = mv2[pl.ds(r * L, L)]
                    y = yv2[pl.ds(r * L, L)]
                    for j in range(JB):
                        sl = pl.ds(d0 + j * L, L)
                        buf[r, sl] = (hbuf[r, sl] - m) * y * gs[j] + bs[j]
                    return c2

                lax.fori_loop(0, RPC, row_body, 0)
                return carry

            lax.fori_loop(0, DB, dblk_body, 0)

        def gather(c):
            hh, b = divmod(c, B)
            return pltpu.async_copy(
                w_hbm.at[idx_v.at[b, pl.ds(hh * RPC, RPC)]],
                rows[c % NBUF], gsem[c % NBUF])

        gath = [None] * NBUF
        outc = [None] * NBUF
        for cp in cp_i:
            cp.wait()
        for c in range(NBUF):
            gath[c] = gather(c)
        cp_g.wait()
        cp_b.wait()

        for c in range(CH):
            cb = c % NBUF
            hh, b = divmod(c, B)
            if c % B == 0:
                cp_p[hh % 2].wait()
                if hh + 1 < NPS:
                    cp_p[(hh + 1) % 2] = pltpu.async_copy(
                        p_hbm.at[pl.ds(pos0 + (hh + 1) * RPC, RPC), :],
                        pbufs[(hh + 1) % 2], psem[(hh + 1) % 2])
            with jax.named_scope(f"gw{c}"):
                gath[cb].wait()
            with jax.named_scope(f"p1_{c}"):
                pass1(rows[cb], pbufs[hh % 2])
            with jax.named_scope(f"p2_{c}"):
                pass2(rows[cb])
            outc[cb] = pltpu.async_copy(
                rows[cb], out_hbm.at[b, pl.ds(pos0 + hh * RPC, RPC), :],
                osem[cb])
            n = c + LOOK
            if NBUF <= n < CH:
                outc[n % NBUF].wait()
                gath[n % NBUF] = gather(n)
        for c in range(CH - NBUF, CH):
            outc[c % NBUF].wait()

    return emb_ln


_emb_ln = _make_kernel()


@jax.jit
def kernel(x, W, P, gamma, beta):
    return _emb_ln(x.astype(jnp.int32), W, P, gamma, beta)


# speedup vs baseline: 1.5000x; 1.5000x over previous
import functools

import jax
import jax.numpy as jnp
from jax import lax
from jax.experimental import pallas as pl
from jax.experimental.pallas import tpu as pltpu
from jax.experimental.pallas import tpu_sc as plsc

B = 4
S = 2048
D = 768
L = 16
NV = D // L

_info = plsc.get_sparse_core_info()
NC = _info.num_cores
NS = _info.num_subcores
NW = NC * NS
PPW = S // NW

RPC = 32
CH = (B * PPW) // RPC
G = RPC // L
NBUF = 3
LOOK = 1
NPS = PPW // RPC
UNJ = 8
JB = 8
DB = D // (JB * L)

_GATHER_DNUMS = lax.GatherDimensionNumbers(
    offset_dims=(), collapsed_slice_dims=(0,), start_index_map=(0,))


def _xlane(x, pm):
    return lax.gather(x, pm[:, None], _GATHER_DNUMS, slice_sizes=(1,),
                      mode=lax.GatherScatterMode.PROMISE_IN_BOUNDS)


def _rsqrt(x):
    iv = lax.bitcast_convert_type(x, jnp.int32)
    iv = jnp.int32(0x5F3759DF) - lax.shift_right_logical(iv, 1)
    y = lax.bitcast_convert_type(iv, jnp.float32)
    for _ in range(3):
        y = y * (1.5 - 0.5 * x * y * y)
    return y


def _tree16(vs, lane):
    level = list(vs)
    for k in range(4):
        bit = 1 << k
        pm = lane ^ bit
        mk = (lane & bit) != 0
        nxt = []
        for j in range(len(level) // 2):
            a, b = level[2 * j], level[2 * j + 1]
            c = (jnp.where(mk, _xlane(b, pm), a)
                 + jnp.where(mk, b, _xlane(a, pm)))
            nxt.append(c)
        level = nxt
    return level[0]


def _make_kernel():
    mesh = plsc.VectorSubcoreMesh(core_axis_name="c", subcore_axis_name="s")

    @functools.partial(
        pl.kernel,
        mesh=mesh,
        out_type=jax.ShapeDtypeStruct((B, S, D), jnp.float32),
        scratch_types=[
            pltpu.VMEM((B, PPW), jnp.int32),
            [pltpu.VMEM((RPC, D), jnp.float32) for _ in range(2)],
            [pltpu.VMEM((RPC, D), jnp.float32) for _ in range(NBUF)],
            pltpu.VMEM((D,), jnp.float32),
            pltpu.VMEM((D,), jnp.float32),
            pltpu.VMEM((L * L,), jnp.float32),
            pltpu.VMEM((L * L,), jnp.float32),
            pltpu.VMEM((RPC * L,), jnp.float32),
            pltpu.VMEM((RPC * L,), jnp.float32),
            [pltpu.SemaphoreType.DMA for _ in range(NBUF)],
            [pltpu.SemaphoreType.DMA for _ in range(NBUF)],
            [pltpu.SemaphoreType.DMA for _ in range(2)],
            pltpu.SemaphoreType.DMA,
            pltpu.SemaphoreType.DMA,
            pltpu.SemaphoreType.DMA,
        ],
    )
    def emb_ln(x_hbm, w_hbm, p_hbm, g_hbm, be_hbm, out_hbm,
               idx_v, pbufs, rows, g_v, be_v, accb, acc2b, mv2, yv2,
               gsem, osem, psem, isem, sgsem, sbsem):
        wid = lax.axis_index("s") * NC + lax.axis_index("c")
        pos0 = wid * PPW

        cp_i = [pltpu.async_copy(x_hbm.at[b, pl.ds(pos0, PPW)],
                                 idx_v.at[b], isem) for b in range(B)]
        cp_p = [None, None]
        cp_p[0] = pltpu.async_copy(p_hbm.at[pl.ds(pos0, RPC), :],
                                   pbufs[0], psem[0])
        cp_g = pltpu.async_copy(g_hbm, g_v, sgsem)
        cp_b = pltpu.async_copy(be_hbm, be_v, sbsem)

        lane = lax.iota(jnp.int32, L)
        zero = jnp.zeros((L,), jnp.float32)

        def pass1(buf, pbuf, k):
            def pair_body(i, carry):
                rb = k * L + 2 * i

                def jblk_body(jc, accs):
                    (a00, a01, a10, a11, b00, b01, b10, b11) = accs
                    acc = [[a00, a01], [a10, a11]]
                    acc2 = [[b00, b01], [b10, b11]]
                    for jj in range(UNJ):
                        sl = pl.ds(jc * (UNJ * L) + jj * L, L)
                        for rr in range(2):
                            v = buf[rb + rr, sl] + pbuf[rb + rr, sl]
                            a = jj % 2
                            acc[rr][a] = acc[rr][a] + v
                            acc2[rr][a] = acc2[rr][a] + v * v
                    return (acc[0][0], acc[0][1], acc[1][0], acc[1][1],
                            acc2[0][0], acc2[0][1], acc2[1][0], acc2[1][1])

                accs = lax.fori_loop(0, NV // UNJ, jblk_body, (zero,) * 8)
                accb[pl.ds(2 * i * L, L)] = accs[0] + accs[1]
                accb[pl.ds((2 * i + 1) * L, L)] = accs[2] + accs[3]
                acc2b[pl.ds(2 * i * L, L)] = accs[4] + accs[5]
                acc2b[pl.ds((2 * i + 1) * L, L)] = accs[6] + accs[7]
                return carry

            lax.fori_loop(0, L // 2, pair_body, 0)
            w = _tree16([accb[pl.ds(t * L, L)] for t in range(L)], lane)
            w2 = _tree16([acc2b[pl.ds(t * L, L)] for t in range(L)], lane)
            mean = w * (1.0 / D)
            var = w2 * (1.0 / D) - mean * mean
            y = _rsqrt(var + 1e-5)
            for t in range(L):
                pm = jnp.full((L,), t, jnp.int32)
                mv2[pl.ds((k * L + t) * L, L)] = _xlane(mean, pm)
                yv2[pl.ds((k * L + t) * L, L)] = _xlane(y, pm)

        def pass2(buf, pbuf):
            def dblk_body(dblk, carry):
                d0 = dblk * (JB * L)
                gs = [g_v[pl.ds(d0 + j * L, L)] for j in range(JB)]
                bs = [be_v[pl.ds(d0 + j * L, L)] for j in range(JB)]

                def row_body(r, c2):
                    m = mv2[pl.ds(r * L, L)]
                    y = yv2[pl.ds(r * L, L)]
                    for j in range(JB):
                        sl = pl.ds(d0 + j * L, L)
                        h = buf[r, sl] + pbuf[r, sl]
                        buf[r, sl] = (h - m) * y * gs[j] + bs[j]
                    return c2

                lax.fori_loop(0, RPC, row_body, 0)
                return carry

            lax.fori_loop(0, DB, dblk_body, 0)

        def gather(c):
            hh, b = divmod(c, B)
            return pltpu.async_copy(
                w_hbm.at[idx_v.at[b, pl.ds(hh * RPC, RPC)]],
                rows[c % NBUF], gsem[c % NBUF])

        gath = [None] * NBUF
        outc = [None] * NBUF
        for cp in cp_i:
            cp.wait()
        for c in range(NBUF):
            gath[c] = gather(c)
        cp_g.wait()
        cp_b.wait()

        for c in range(CH):
            cb = c % NBUF
            hh, b = divmod(c, B)
            if c % B == 0:
                cp_p[hh % 2].wait()
                if hh + 1 < NPS:
                    cp_p[(hh + 1) % 2] = pltpu.async_copy(
                        p_hbm.at[pl.ds(pos0 + (hh + 1) * RPC, RPC), :],
                        pbufs[(hh + 1) % 2], psem[(hh + 1) % 2])
            gath[cb].wait()
            for k in range(G):
                pass1(rows[cb], pbufs[hh % 2], k)
            pass2(rows[cb], pbufs[hh % 2])
            outc[cb] = pltpu.async_copy(
                rows[cb], out_hbm.at[b, pl.ds(pos0 + hh * RPC, RPC), :],
                osem[cb])
            n = c + LOOK
            if NBUF <= n < CH:
                outc[n % NBUF].wait()
                gath[n % NBUF] = gather(n)
        for c in range(CH - NBUF, CH):
            outc[c % NBUF].wait()

    return emb_ln


_emb_ln = _make_kernel()


@jax.jit
def kernel(x, W, P, gamma, beta):
    return _emb_ln(x.astype(jnp.int32), W, P, gamma, beta)


# speedup vs baseline: 1.7143x; 1.1428x over previous
import functools

import jax
import jax.numpy as jnp
from jax import lax
from jax.experimental import pallas as pl
from jax.experimental.pallas import tpu as pltpu
from jax.experimental.pallas import tpu_sc as plsc

B = 4
S = 2048
D = 768
L = 16
NV = D // L

_info = plsc.get_sparse_core_info()
NC = _info.num_cores
NS = _info.num_subcores
NW = NC * NS
PPW = S // NW

RPC = 16
CH = (B * PPW) // RPC
NBUF = 8
LOOK = 4
NPS = PPW // RPC
UNJ = 8
JB = 8
DB = D // (JB * L)

_GATHER_DNUMS = lax.GatherDimensionNumbers(
    offset_dims=(), collapsed_slice_dims=(0,), start_index_map=(0,))


def _xlane(x, pm):
    return lax.gather(x, pm[:, None], _GATHER_DNUMS, slice_sizes=(1,),
                      mode=lax.GatherScatterMode.PROMISE_IN_BOUNDS)


def _rsqrt(x):
    iv = lax.bitcast_convert_type(x, jnp.int32)
    iv = jnp.int32(0x5F3759DF) - lax.shift_right_logical(iv, 1)
    y = lax.bitcast_convert_type(iv, jnp.float32)
    for _ in range(3):
        y = y * (1.5 - 0.5 * x * y * y)
    return y


def _tree16(vs, lane):
    level = list(vs)
    for k in range(4):
        bit = 1 << k
        pm = lane ^ bit
        mk = (lane & bit) != 0
        nxt = []
        for j in range(len(level) // 2):
            a, b = level[2 * j], level[2 * j + 1]
            c = (jnp.where(mk, _xlane(b, pm), a)
                 + jnp.where(mk, b, _xlane(a, pm)))
            nxt.append(c)
        level = nxt
    return level[0]


def _make_kernel():
    mesh = plsc.VectorSubcoreMesh(core_axis_name="c", subcore_axis_name="s")

    @functools.partial(
        pl.kernel,
        mesh=mesh,
        out_type=jax.ShapeDtypeStruct((B, S, D), jnp.float32),
        scratch_types=[
            pltpu.VMEM((B, PPW), jnp.int32),
            [pltpu.VMEM((RPC, D), jnp.float32) for _ in range(2)],
            [pltpu.VMEM((RPC, D), jnp.float32) for _ in range(NBUF)],
            pltpu.VMEM((D,), jnp.float32),
            pltpu.VMEM((D,), jnp.float32),
            pltpu.VMEM((L * L,), jnp.float32),
            pltpu.VMEM((L * L,), jnp.float32),
            pltpu.VMEM((L * L,), jnp.float32),
            pltpu.VMEM((L * L,), jnp.float32),
            [pltpu.SemaphoreType.DMA for _ in range(NBUF)],
            [pltpu.SemaphoreType.DMA for _ in range(NBUF)],
            [pltpu.SemaphoreType.DMA for _ in range(2)],
            pltpu.SemaphoreType.DMA,
            pltpu.SemaphoreType.DMA,
            pltpu.SemaphoreType.DMA,
        ],
    )
    def emb_ln(x_hbm, w_hbm, p_hbm, g_hbm, be_hbm, out_hbm,
               idx_v, pbufs, rows, g_v, be_v, accb, acc2b, mv2, yv2,
               gsem, osem, psem, isem, sgsem, sbsem):
        wid = lax.axis_index("s") * NC + lax.axis_index("c")
        pos0 = wid * PPW

        cp_i = [pltpu.async_copy(x_hbm.at[b, pl.ds(pos0, PPW)],
                                 idx_v.at[b], isem) for b in range(B)]
        cp_p = [None, None]
        cp_p[0] = pltpu.async_copy(p_hbm.at[pl.ds(pos0, RPC), :],
                                   pbufs[0], psem[0])
        cp_g = pltpu.async_copy(g_hbm, g_v, sgsem)
        cp_b = pltpu.async_copy(be_hbm, be_v, sbsem)

        lane = lax.iota(jnp.int32, L)
        zero = jnp.zeros((L,), jnp.float32)

        def pass1(buf, pbuf):
            def pair_body(i, carry):
                rb = 2 * i

                def jblk_body(jc, accs):
                    (a00, a01, a10, a11, b00, b01, b10, b11) = accs
                    acc = [[a00, a01], [a10, a11]]
                    acc2 = [[b00, b01], [b10, b11]]
                    for jj in range(UNJ):
                        sl = pl.ds(jc * (UNJ * L) + jj * L, L)
                        for rr in range(2):
                            v = buf[rb + rr, sl] + pbuf[rb + rr, sl]
                            a = jj % 2
                            acc[rr][a] = acc[rr][a] + v
                            acc2[rr][a] = acc2[rr][a] + v * v
                    return (acc[0][0], acc[0][1], acc[1][0], acc[1][1],
                            acc2[0][0], acc2[0][1], acc2[1][0], acc2[1][1])

                accs = lax.fori_loop(0, NV // UNJ, jblk_body, (zero,) * 8)
                accb[pl.ds(2 * i * L, L)] = accs[0] + accs[1]
                accb[pl.ds((2 * i + 1) * L, L)] = accs[2] + accs[3]
                acc2b[pl.ds(2 * i * L, L)] = accs[4] + accs[5]
                acc2b[pl.ds((2 * i + 1) * L, L)] = accs[6] + accs[7]
                return carry

            lax.fori_loop(0, L // 2, pair_body, 0)
            w = _tree16([accb[pl.ds(t * L, L)] for t in range(L)], lane)
            w2 = _tree16([acc2b[pl.ds(t * L, L)] for t in range(L)], lane)
            mean = w * (1.0 / D)
            var = w2 * (1.0 / D) - mean * mean
            y = _rsqrt(var + 1e-5)
            for t in range(L):
                pm = jnp.full((L,), t, jnp.int32)
                mv2[pl.ds(t * L, L)] = _xlane(mean, pm)
                yv2[pl.ds(t * L, L)] = _xlane(y, pm)

        def pass2(buf, pbuf):
            def dblk_body(dblk, carry):
                d0 = dblk * (JB * L)
                gs = [g_v[pl.ds(d0 + j * L, L)] for j in range(JB)]
                bs = [be_v[pl.ds(d0 + j * L, L)] for j in range(JB)]

                def row_body(r, c2):
                    m = mv2[pl.ds(r * L, L)]
                    y = yv2[pl.ds(r * L, L)]
                    for j in range(JB):
                        sl = pl.ds(d0 + j * L, L)
                        h = buf[r, sl] + pbuf[r, sl]
                        buf[r, sl] = (h - m) * y * gs[j] + bs[j]
                    return c2

                lax.fori_loop(0, RPC, row_body, 0)
                return carry

            lax.fori_loop(0, DB, dblk_body, 0)

        def gather(c):
            hh, b = divmod(c, B)
            return pltpu.async_copy(
                w_hbm.at[idx_v.at[b, pl.ds(hh * RPC, RPC)]],
                rows[c % NBUF], gsem[c % NBUF])

        gath = [None] * NBUF
        outc = [None] * NBUF
        for cp in cp_i:
            cp.wait()
        for c in range(NBUF):
            gath[c] = gather(c)
        cp_g.wait()
        cp_b.wait()

        for c in range(CH):
            cb = c % NBUF
            hh, b = divmod(c, B)
            if c % B == 0:
                cp_p[hh % 2].wait()
                if hh + 1 < NPS:
                    cp_p[(hh + 1) % 2] = pltpu.async_copy(
                        p_hbm.at[pl.ds(pos0 + (hh + 1) * RPC, RPC), :],
                        pbufs[(hh + 1) % 2], psem[(hh + 1) % 2])
            gath[cb].wait()
            pass1(rows[cb], pbufs[hh % 2])
            pass2(rows[cb], pbufs[hh % 2])
            outc[cb] = pltpu.async_copy(
                rows[cb], out_hbm.at[b, pl.ds(pos0 + hh * RPC, RPC), :],
                osem[cb])
            n = c + LOOK
            if NBUF <= n < CH:
                outc[n % NBUF].wait()
                gath[n % NBUF] = gather(n)
        for c in range(CH - NBUF, CH):
            outc[c % NBUF].wait()

    return emb_ln


_emb_ln = _make_kernel()


@jax.jit
def kernel(x, W, P, gamma, beta):
    return _emb_ln(x.astype(jnp.int32), W, P, gamma, beta)
